# baseline (speedup 1.0000x reference)
import functools

import jax
import jax.numpy as jnp
from jax import lax
from jax.experimental import pallas as pl
from jax.experimental.pallas import tpu as pltpu
from jax.experimental.pallas import tpu_sc as plsc

N_EXPERT = 16
DIM_IN = 2048
TILE = 1024

_NC = 2
_NS = 16
_NW = _NC * _NS


def _logits_kernel(x_ref, wt_ref, out_ref):
    out_ref[...] = jax.lax.dot_general(
        x_ref[...], wt_ref[...], (((1,), (0,)), ((), ())),
        preferred_element_type=jnp.float32,
    )


def _make_sc_gating(tokens):
    t_per_w = tokens // _NW
    mesh = plsc.VectorSubcoreMesh(core_axis_name="c", subcore_axis_name="s")

    @functools.partial(
        pl.kernel,
        mesh=mesh,
        out_type=[
            jax.ShapeDtypeStruct((tokens, N_EXPERT), jnp.float32),
            jax.ShapeDtypeStruct((tokens, N_EXPERT), jnp.int32),
        ],
        scratch_types=[
            pltpu.VMEM((t_per_w, N_EXPERT), jnp.float32),
            pltpu.VMEM((t_per_w, N_EXPERT), jnp.float32),
            pltpu.VMEM((t_per_w, N_EXPERT), jnp.int32),
        ],
    )
    def gate(logits_hbm, cw_hbm, ei_hbm, rows_v, cw_v, ei_v):
        wid = lax.axis_index("s") * _NC + lax.axis_index("c")
        base = wid * t_per_w
        pltpu.sync_copy(logits_hbm.at[pl.ds(base, t_per_w)], rows_v)

        iota = lax.iota(jnp.int32, N_EXPERT)
        perms = [iota ^ s for s in (1, 2, 4, 8)]

        def allreduce(v, op):
            for p in perms:
                v = op(v, v.at[p].get(mode="promise_in_bounds"))
            return v

        def body(i, carry):
            l = rows_v[i]
            m1 = allreduce(l, jnp.maximum)
            idx1 = allreduce(
                jnp.where(l == m1, iota, N_EXPERT), jnp.minimum
            )
            masked = jnp.where(iota == idx1, -jnp.inf, l)
            m2 = allreduce(masked, jnp.maximum)
            idx2 = allreduce(
                jnp.where(masked == m2, iota, N_EXPERT), jnp.minimum
            )
            z = allreduce(jnp.exp(l - m1), jnp.add)
            p1 = 1.0 / z
            p2 = jnp.exp(m2 - m1) / z
            den = p1 + p2 + 1e-09
            lane0 = iota == 0
            lane1 = iota == 1
            cw_v[i] = jnp.where(
                lane0, p1 / den, jnp.where(lane1, p2 / den, 0.0)
            )
            ei_v[i] = jnp.where(lane0, idx1, jnp.where(lane1, idx2, 0))
            return carry

        lax.fori_loop(0, t_per_w, body, 0)
        pltpu.sync_copy(cw_v, cw_hbm.at[pl.ds(base, t_per_w)])
        pltpu.sync_copy(ei_v, ei_hbm.at[pl.ds(base, t_per_w)])

    return gate


def kernel(x, W):
    b, n, d = x.shape
    tokens = b * n
    xf = x.reshape(tokens, d)
    wt = W.T
    logits = pl.pallas_call(
        _logits_kernel,
        grid=(tokens // TILE,),
        in_specs=[
            pl.BlockSpec((TILE, d), lambda i: (i, 0)),
            pl.BlockSpec((d, N_EXPERT), lambda i: (0, 0)),
        ],
        out_specs=pl.BlockSpec((TILE, N_EXPERT), lambda i: (i, 0)),
        out_shape=jax.ShapeDtypeStruct((tokens, N_EXPERT), jnp.float32),
        compiler_params=pltpu.CompilerParams(
            dimension_semantics=("parallel",),
        ),
    )(xf, wt)
    cw_pad, ei_pad = _make_sc_gating(tokens)(logits)
    return (
        cw_pad[:, :2].reshape(b, n, 2),
        ei_pad[:, :2].reshape(b, n, 2),
    )


# speedup vs baseline: 1.8016x; 1.8016x over previous
import jax
import jax.numpy as jnp
from jax.experimental import pallas as pl
from jax.experimental.pallas import tpu as pltpu

N_EXPERT = 16
DIM_IN = 2048
TILE = 1024


def _gating_kernel(x_ref, w_ref, cw_ref, ei_ref):
    x = x_ref[...]
    w = w_ref[...]
    logits = jax.lax.dot_general(
        x, w, (((1,), (1,)), ((), ())), preferred_element_type=jnp.float32
    )
    lt = logits.T
    t = lt.shape[1]
    iota = jax.lax.broadcasted_iota(jnp.int32, (N_EXPERT, t), 0)

    m1 = jnp.max(lt, axis=0, keepdims=True)
    idx1 = jnp.min(
        jnp.where(lt == m1, iota, N_EXPERT), axis=0, keepdims=True
    )
    masked = jnp.where(iota == idx1, -jnp.inf, lt)
    m2 = jnp.max(masked, axis=0, keepdims=True)
    idx2 = jnp.min(
        jnp.where(masked == m2, iota, N_EXPERT), axis=0, keepdims=True
    )

    z = jnp.sum(jnp.exp(lt - m1), axis=0, keepdims=True)
    p1 = 1.0 / z
    p2 = jnp.exp(m2 - m1) / z
    den = p1 + p2 + 1e-09
    cwt = jnp.concatenate([p1 / den, p2 / den], axis=0)
    eit = jnp.concatenate([idx1, idx2], axis=0)
    cw_ref[...] = cwt.T
    ei_ref[...] = eit.T


def kernel(x, W):
    b, n, d = x.shape
    tokens = b * n
    xf = x.reshape(tokens, d)
    grid = (tokens // TILE,)
    cw, ei = pl.pallas_call(
        _gating_kernel,
        grid=grid,
        in_specs=[
            pl.BlockSpec((TILE, d), lambda i: (i, 0)),
            pl.BlockSpec((N_EXPERT, d), lambda i: (0, 0)),
        ],
        out_specs=[
            pl.BlockSpec((TILE, 2), lambda i: (i, 0)),
            pl.BlockSpec((TILE, 2), lambda i: (i, 0)),
        ],
        out_shape=[
            jax.ShapeDtypeStruct((tokens, 2), jnp.float32),
            jax.ShapeDtypeStruct((tokens, 2), jnp.int32),
        ],
        compiler_params=pltpu.CompilerParams(
            dimension_semantics=("parallel",),
        ),
    )(xf, W)
    return cw.reshape(b, n, 2), ei.reshape(b, n, 2)
